# baseline (speedup 1.0000x reference)
import functools

import jax
import jax.numpy as jnp
from jax import lax
from jax.experimental import pallas as pl
from jax.experimental.pallas import tpu as pltpu

_B = 8
_R = 29
_K = 2048
_H = 768
_KB = 512
_KC = _K // _KB


def _region_body(q_ref, te_ref, sim_ref, idx_ref, acc_sim, acc_idx):
    kc = pl.program_id(1)
    q = q_ref[0]
    te = te_ref[0]
    qn = q / jnp.maximum(jnp.sqrt(jnp.sum(q * q, axis=1, keepdims=True)), 1e-12)
    tn = te / jnp.maximum(jnp.sqrt(jnp.sum(te * te, axis=1, keepdims=True)), 1e-12)
    sims = lax.dot_general(qn, tn, (((1,), (1,)), ((), ())),
                           preferred_element_type=jnp.float32)
    lbest = jnp.max(sims, axis=1)
    kiota = lax.broadcasted_iota(jnp.int32, (_B, _KB), 1)
    lidx = jnp.min(jnp.where(sims == lbest[:, None], kiota, _KB),
                   axis=1) + kc * _KB

    @pl.when(kc == 0)
    def _init():
        acc_sim[0] = lbest
        acc_idx[0] = lidx

    @pl.when(kc > 0)
    def _merge():
        take = lbest > acc_sim[0]
        acc_sim[0] = jnp.where(take, lbest, acc_sim[0])
        acc_idx[0] = jnp.where(take, lidx, acc_idx[0])

    @pl.when(kc == _KC - 1)
    def _flush():
        sim_ref[0, 0] = acc_sim[0]
        idx_ref[0, 0] = acc_idx[0]


@jax.jit
def _retrieve(vf_regions, text_embeddings):
    sim, idx = pl.pallas_call(
        _region_body,
        grid=(_R, _KC),
        in_specs=[
            pl.BlockSpec((1, _B, _H), lambda r, kc: (r, 0, 0)),
            pl.BlockSpec((1, _KB, _H), lambda r, kc: (r, kc, 0)),
        ],
        out_specs=[
            pl.BlockSpec((1, 1, _B), lambda r, kc: (r, 0, 0)),
            pl.BlockSpec((1, 1, _B), lambda r, kc: (r, 0, 0)),
        ],
        out_shape=[
            jax.ShapeDtypeStruct((_R, 1, _B), jnp.float32),
            jax.ShapeDtypeStruct((_R, 1, _B), jnp.int32),
        ],
        scratch_shapes=[
            pltpu.VMEM((1, _B), jnp.float32),
            pltpu.VMEM((1, _B), jnp.int32),
        ],
        compiler_params=pltpu.CompilerParams(
            dimension_semantics=("arbitrary", "arbitrary"),
        ),
    )(vf_regions, text_embeddings)
    return sim, idx


def kernel(visual_features, text_embeddings):
    vf_regions = jnp.transpose(visual_features[:, 1:1 + _R, :], (1, 0, 2))
    sim, idx = _retrieve(vf_regions, text_embeddings)
    best_sim = jnp.transpose(sim.reshape(_R, _B), (1, 0))
    best_idx = jnp.transpose(idx.reshape(_R, _B), (1, 0))
    return best_sim, best_idx


# speedup vs baseline: 1.9195x; 1.9195x over previous
import functools

import jax
import jax.numpy as jnp
from jax import lax
from jax.experimental import pallas as pl
from jax.experimental.pallas import tpu as pltpu

_B = 8
_R = 29
_K = 2048
_H = 768
_KB = 1024


def _best_of(sims, base):
    lbest = jnp.max(sims, axis=1)
    kiota = lax.broadcasted_iota(jnp.int32, (_B, _KB), 1)
    lidx = jnp.min(jnp.where(sims == lbest[:, None], kiota, _KB),
                   axis=1) + base
    return lbest, lidx


def _region_body(q_ref, te0_ref, te1_ref, sim_ref, idx_ref):
    q = q_ref[0]
    qn = q / jnp.maximum(jnp.sqrt(jnp.sum(q * q, axis=1, keepdims=True)), 1e-12)

    def sims_of(te):
        tn = te / jnp.maximum(
            jnp.sqrt(jnp.sum(te * te, axis=1, keepdims=True)), 1e-12)
        return lax.dot_general(qn, tn, (((1,), (1,)), ((), ())),
                               preferred_element_type=jnp.float32)

    b0, i0 = _best_of(sims_of(te0_ref[0]), 0)
    b1, i1 = _best_of(sims_of(te1_ref[0]), _KB)
    take = b1 > b0
    sim_ref[0, 0] = jnp.where(take, b1, b0)
    idx_ref[0, 0] = jnp.where(take, i1, i0)


@jax.jit
def _retrieve(vf_regions, text_embeddings):
    sim, idx = pl.pallas_call(
        _region_body,
        grid=(_R,),
        in_specs=[
            pl.BlockSpec((1, _B, _H), lambda r: (r, 0, 0)),
            pl.BlockSpec((1, _KB, _H), lambda r: (r, 0, 0)),
            pl.BlockSpec((1, _KB, _H), lambda r: (r, 1, 0)),
        ],
        out_specs=[
            pl.BlockSpec((1, 1, _B), lambda r: (r, 0, 0)),
            pl.BlockSpec((1, 1, _B), lambda r: (r, 0, 0)),
        ],
        out_shape=[
            jax.ShapeDtypeStruct((_R, 1, _B), jnp.float32),
            jax.ShapeDtypeStruct((_R, 1, _B), jnp.int32),
        ],
        compiler_params=pltpu.CompilerParams(
            dimension_semantics=("arbitrary",),
        ),
    )(vf_regions, text_embeddings, text_embeddings)
    return sim, idx


def kernel(visual_features, text_embeddings):
    vf_regions = jnp.transpose(visual_features[:, 1:1 + _R, :], (1, 0, 2))
    sim, idx = _retrieve(vf_regions, text_embeddings)
    best_sim = jnp.transpose(sim.reshape(_R, _B), (1, 0))
    best_idx = jnp.transpose(idx.reshape(_R, _B), (1, 0))
    return best_sim, best_idx


# speedup vs baseline: 1.9600x; 1.0211x over previous
import functools

import jax
import jax.numpy as jnp
from jax import lax
from jax.experimental import pallas as pl
from jax.experimental.pallas import tpu as pltpu

_B = 8
_R = 29
_K = 2048
_H = 768
_KB = 512


def _best_of(sims, base):
    lbest = jnp.max(sims, axis=1)
    kiota = lax.broadcasted_iota(jnp.int32, (_B, _KB), 1)
    lidx = jnp.min(jnp.where(sims == lbest[:, None], kiota, _KB),
                   axis=1) + base
    return lbest, lidx


def _region_body(q_ref, te0_ref, te1_ref, te2_ref, te3_ref, sim_ref, idx_ref):
    q = q_ref[0]
    qn = q / jnp.maximum(jnp.sqrt(jnp.sum(q * q, axis=1, keepdims=True)), 1e-12)

    def sims_of(te):
        tn = te / jnp.maximum(
            jnp.sqrt(jnp.sum(te * te, axis=1, keepdims=True)), 1e-12)
        return lax.dot_general(qn, tn, (((1,), (1,)), ((), ())),
                               preferred_element_type=jnp.float32)

    best, bidx = _best_of(sims_of(te0_ref[0]), 0)
    for s, ref in enumerate((te1_ref, te2_ref, te3_ref)):
        b, i = _best_of(sims_of(ref[0]), (s + 1) * _KB)
        take = b > best
        best = jnp.where(take, b, best)
        bidx = jnp.where(take, i, bidx)
    sim_ref[0, 0] = best
    idx_ref[0, 0] = bidx


@jax.jit
def _retrieve(vf_regions, text_embeddings):
    sim, idx = pl.pallas_call(
        _region_body,
        grid=(_R,),
        in_specs=[
            pl.BlockSpec((1, _B, _H), lambda r: (r, 0, 0)),
            pl.BlockSpec((1, _KB, _H), lambda r: (r, 0, 0)),
            pl.BlockSpec((1, _KB, _H), lambda r: (r, 1, 0)),
            pl.BlockSpec((1, _KB, _H), lambda r: (r, 2, 0)),
            pl.BlockSpec((1, _KB, _H), lambda r: (r, 3, 0)),
        ],
        out_specs=[
            pl.BlockSpec((1, 1, _B), lambda r: (r, 0, 0)),
            pl.BlockSpec((1, 1, _B), lambda r: (r, 0, 0)),
        ],
        out_shape=[
            jax.ShapeDtypeStruct((_R, 1, _B), jnp.float32),
            jax.ShapeDtypeStruct((_R, 1, _B), jnp.int32),
        ],
        compiler_params=pltpu.CompilerParams(
            dimension_semantics=("arbitrary",),
        ),
    )(vf_regions, text_embeddings, text_embeddings, text_embeddings, text_embeddings)
    return sim, idx


def kernel(visual_features, text_embeddings):
    vf_regions = jnp.transpose(visual_features[:, 1:1 + _R, :], (1, 0, 2))
    sim, idx = _retrieve(vf_regions, text_embeddings)
    best_sim = jnp.transpose(sim.reshape(_R, _B), (1, 0))
    best_idx = jnp.transpose(idx.reshape(_R, _B), (1, 0))
    return best_sim, best_idx
